# Initial kernel scaffold; baseline (speedup 1.0000x reference)
import functools

import jax
import jax.numpy as jnp
from jax import lax
from jax.experimental import pallas as pl
from jax.experimental.pallas import tpu as pltpu
from jax.experimental.pallas import tpu_sc as plsc

N = 10000
E = 320000
D = 128
H = 256
NC = 2
NS = 16
NW = NC * NS
CHUNK = 128
NCHUNK = 80
PER_TILE = NCHUNK * CHUNK
E_PAD = NW * PER_TILE
ACC_N = 10240
SLAB = ACC_N // NS
HW = 16
EPS = 1e-5

_mesh = functools.partial(
    plsc.VectorSubcoreMesh, core_axis_name="c", subcore_axis_name="s"
)


def _sc_hist(dst_idx, ones_rows, zeros_hist):
    @functools.partial(
        pl.kernel,
        out_type=jax.ShapeDtypeStruct((NC, ACC_N, HW), jnp.float32),
        mesh=_mesh(),
        scratch_types=[
            pltpu.VMEM((NCHUNK, CHUNK), jnp.int32),
            pltpu.VMEM((CHUNK, HW), jnp.float32),
            pltpu.VMEM_SHARED((ACC_N, HW), jnp.float32),
        ],
    )
    def k(dst_hbm, ones_hbm, zeros_hbm, out_hbm, dst_v, ones_v, acc):
        c = lax.axis_index("c")
        s = lax.axis_index("s")
        wid = s * NC + c
        pltpu.sync_copy(
            zeros_hbm.at[pl.ds(s * SLAB, SLAB)], acc.at[pl.ds(s * SLAB, SLAB)]
        )
        pltpu.sync_copy(dst_hbm.at[wid], dst_v)
        pltpu.sync_copy(ones_hbm, ones_v)
        plsc.subcore_barrier()

        @pl.loop(0, NCHUNK)
        def _(j):
            pltpu.sync_copy(ones_v, acc.at[dst_v.at[j]], add=True)

        plsc.subcore_barrier()
        pltpu.sync_copy(
            acc.at[pl.ds(s * SLAB, SLAB)], out_hbm.at[c].at[pl.ds(s * SLAB, SLAB)]
        )

    return k(dst_idx, ones_rows, zeros_hist)


def _sc_agg(table, src_idx, dst_idx, zeros_acc):
    @functools.partial(
        pl.kernel,
        out_type=jax.ShapeDtypeStruct((NC, ACC_N, D), jnp.float32),
        mesh=_mesh(),
        scratch_types=[
            pltpu.VMEM((NCHUNK, CHUNK), jnp.int32),
            pltpu.VMEM((NCHUNK, CHUNK), jnp.int32),
            pltpu.VMEM((CHUNK, D), jnp.float32),
            pltpu.VMEM_SHARED((ACC_N, D), jnp.float32),
            pltpu.SemaphoreType.DMA,
        ],
    )
    def k(table_hbm, src_hbm, dst_hbm, zeros_hbm, out_hbm,
          src_v, dst_v, rows_v, acc, sem):
        c = lax.axis_index("c")
        s = lax.axis_index("s")
        wid = s * NC + c
        pltpu.sync_copy(
            zeros_hbm.at[pl.ds(s * SLAB, SLAB)], acc.at[pl.ds(s * SLAB, SLAB)]
        )
        pltpu.sync_copy(src_hbm.at[wid], src_v)
        pltpu.sync_copy(dst_hbm.at[wid], dst_v)
        plsc.subcore_barrier()

        @pl.loop(0, NCHUNK)
        def _(j):
            pltpu.async_copy(table_hbm.at[src_v.at[j]], rows_v, sem).wait()
            pltpu.sync_copy(rows_v, acc.at[dst_v.at[j]], add=True)

        plsc.subcore_barrier()
        pltpu.sync_copy(
            acc.at[pl.ds(s * SLAB, SLAB)], out_hbm.at[c].at[pl.ds(s * SLAB, SLAB)]
        )

    return k(table, src_idx, dst_idx, zeros_acc)


def _tc_prep(hist2, x):
    def body(hist2_ref, x_ref, xp_ref, dinv_ref, invc_ref):
        hv = hist2_ref[...]
        hist = (hv[0] + hv[1])[:, 0:1]
        dinv = lax.rsqrt(hist + 1.0)
        invc = 1.0 / jnp.maximum(hist, 1.0)
        dinv_ref[...] = dinv
        invc_ref[...] = invc
        xp_ref[0:N, :] = dinv[0:N] * x_ref[...]
        xp_ref[N:ACC_N, :] = jnp.zeros((ACC_N - N, D), jnp.float32)

    return pl.pallas_call(
        body,
        out_shape=(
            jax.ShapeDtypeStruct((ACC_N, D), jnp.float32),
            jax.ShapeDtypeStruct((ACC_N, 1), jnp.float32),
            jax.ShapeDtypeStruct((ACC_N, 1), jnp.float32),
        ),
    )(hist2, x)


def _tc_layer1(S2, xp, dinv, W1, b1, g1, beta1, Wl):
    def body(S2_ref, xp_ref, dinv_ref, W1_ref, b1_ref, g1_ref, beta1_ref,
             Wl_ref, h2_ref, p_ref):
        Sv = S2_ref[...]
        S = Sv[0, 0:N] + Sv[1, 0:N]
        z = dinv_ref[0:N] * (S + xp_ref[0:N])
        m1 = lax.dot_general(
            z, W1_ref[...], (((1,), (1,)), ((), ())),
            preferred_element_type=jnp.float32,
        ) + b1_ref[...][None, :]
        mu = jnp.mean(m1, axis=0, keepdims=True)
        var = jnp.mean(m1 * m1, axis=0, keepdims=True) - mu * mu
        h2 = (m1 - mu) * lax.rsqrt(var + EPS) * g1_ref[...][None, :] \
            + beta1_ref[...][None, :]
        h2 = jnp.maximum(h2, 0.0)
        h2_ref[...] = h2
        p_ref[0:N, :] = lax.dot_general(
            h2, Wl_ref[...], (((1,), (1,)), ((), ())),
            preferred_element_type=jnp.float32,
        )
        p_ref[N:ACC_N, :] = jnp.zeros((ACC_N - N, D), jnp.float32)

    return pl.pallas_call(
        body,
        out_shape=(
            jax.ShapeDtypeStruct((N, H), jnp.float32),
            jax.ShapeDtypeStruct((ACC_N, D), jnp.float32),
        ),
    )(S2, xp, dinv, W1, b1, g1, beta1, Wl)


def _tc_right(h2, Wr):
    def body(h2_ref, Wr_ref, q_ref):
        q_ref[...] = lax.dot_general(
            h2_ref[...], Wr_ref[...], (((1,), (1,)), ((), ())),
            preferred_element_type=jnp.float32,
        )

    return pl.pallas_call(
        body, out_shape=jax.ShapeDtypeStruct((N, D), jnp.float32)
    )(h2, Wr)


def _tc_layer2(T2, invc, q, bl, g2, beta2):
    def body(T2_ref, invc_ref, q_ref, bl_ref, g2_ref, beta2_ref, out_ref):
        Tv = T2_ref[...]
        T = Tv[0, 0:N] + Tv[1, 0:N]
        r = T * invc_ref[0:N] + bl_ref[...][None, :] + q_ref[...]
        mu = jnp.mean(r, axis=0, keepdims=True)
        var = jnp.mean(r * r, axis=0, keepdims=True) - mu * mu
        out = (r - mu) * lax.rsqrt(var + EPS) * g2_ref[...][None, :] \
            + beta2_ref[...][None, :]
        out_ref[...] = jnp.maximum(out, 0.0)

    return pl.pallas_call(
        body, out_shape=jax.ShapeDtypeStruct((N, D), jnp.float32)
    )(T2, invc, q, bl, g2, beta2)


def kernel(x, edge_index, W1, b1, g1, beta1, Wl, bl, Wr, g2, beta2):
    src = edge_index[0].astype(jnp.int32)
    dst = edge_index[1].astype(jnp.int32)
    pad = E_PAD - E
    src_p = jnp.concatenate([src, jnp.full((pad,), N, jnp.int32)])
    dst_p = jnp.concatenate([dst, jnp.full((pad,), N, jnp.int32)])
    src_p = src_p.reshape(NW, NCHUNK, CHUNK)
    dst_p = dst_p.reshape(NW, NCHUNK, CHUNK)

    ones_rows = jnp.ones((CHUNK, HW), jnp.float32)
    zeros_hist = jnp.zeros((ACC_N, HW), jnp.float32)
    zeros_acc = jnp.zeros((ACC_N, D), jnp.float32)

    hist2 = _sc_hist(dst_p, ones_rows, zeros_hist)
    xp, dinv, invc = _tc_prep(hist2, x)
    S2 = _sc_agg(xp, src_p, dst_p, zeros_acc)
    h2, p = _tc_layer1(S2, xp, dinv, W1, b1, g1, beta1, Wl)
    q = _tc_right(h2, Wr)
    T2 = _sc_agg(p, src_p, dst_p, zeros_acc)
    return _tc_layer2(T2, invc, q, bl, g2, beta2)


# speedup vs baseline: 7.8829x; 7.8829x over previous
import functools

import jax
import jax.numpy as jnp
from jax import lax
from jax.experimental import pallas as pl
from jax.experimental.pallas import tpu as pltpu
from jax.experimental.pallas import tpu_sc as plsc

N = 10000
E = 320000
D = 128
H = 256
NC = 2
NS = 16
NW = NC * NS
CHUNK = 128
NCHUNK = 80
PER_TILE = NCHUNK * CHUNK
E_PAD = NW * PER_TILE
ACC_N = 10240
SLAB = ACC_N // NS
HW = 128
EPS = 1e-5

_mesh = functools.partial(
    plsc.VectorSubcoreMesh, core_axis_name="c", subcore_axis_name="s"
)


def _sc_hist(dst_idx, ones_rows, zeros_hist):
    @functools.partial(
        pl.kernel,
        out_type=jax.ShapeDtypeStruct((NC, ACC_N, HW), jnp.float32),
        mesh=_mesh(),
        scratch_types=[
            pltpu.VMEM((NCHUNK, CHUNK), jnp.int32),
            pltpu.VMEM((CHUNK, HW), jnp.float32),
            pltpu.VMEM_SHARED((ACC_N, HW), jnp.float32),
        ],
    )
    def k(dst_hbm, ones_hbm, zeros_hbm, out_hbm, dst_v, ones_v, acc):
        c = lax.axis_index("c")
        s = lax.axis_index("s")
        wid = s * NC + c
        pltpu.sync_copy(
            zeros_hbm.at[pl.ds(s * SLAB, SLAB)], acc.at[pl.ds(s * SLAB, SLAB)]
        )
        pltpu.sync_copy(dst_hbm.at[wid], dst_v)
        pltpu.sync_copy(ones_hbm, ones_v)
        plsc.subcore_barrier()

        @pl.loop(0, NCHUNK)
        def _(j):
            pltpu.sync_copy(ones_v, acc.at[dst_v.at[j]], add=True)

        plsc.subcore_barrier()
        pltpu.sync_copy(
            acc.at[pl.ds(s * SLAB, SLAB)], out_hbm.at[c].at[pl.ds(s * SLAB, SLAB)]
        )

    return k(dst_idx, ones_rows, zeros_hist)


def _sc_agg(table, src_idx, dst_idx, zeros_acc):
    @functools.partial(
        pl.kernel,
        out_type=jax.ShapeDtypeStruct((NC, ACC_N, D), jnp.float32),
        mesh=_mesh(),
        scratch_types=[
            pltpu.VMEM((NCHUNK, CHUNK), jnp.int32),
            pltpu.VMEM((NCHUNK, CHUNK), jnp.int32),
            pltpu.VMEM((CHUNK, D), jnp.float32),
            pltpu.VMEM_SHARED((ACC_N, D), jnp.float32),
            pltpu.SemaphoreType.DMA,
        ],
    )
    def k(table_hbm, src_hbm, dst_hbm, zeros_hbm, out_hbm,
          src_v, dst_v, rows_v, acc, sem):
        c = lax.axis_index("c")
        s = lax.axis_index("s")
        wid = s * NC + c
        pltpu.sync_copy(
            zeros_hbm.at[pl.ds(s * SLAB, SLAB)], acc.at[pl.ds(s * SLAB, SLAB)]
        )
        pltpu.sync_copy(src_hbm.at[wid], src_v)
        pltpu.sync_copy(dst_hbm.at[wid], dst_v)
        plsc.subcore_barrier()

        @pl.loop(0, NCHUNK)
        def _(j):
            pltpu.async_copy(table_hbm.at[src_v.at[j]], rows_v, sem).wait()
            pltpu.sync_copy(rows_v, acc.at[dst_v.at[j]], add=True)

        plsc.subcore_barrier()
        pltpu.sync_copy(
            acc.at[pl.ds(s * SLAB, SLAB)], out_hbm.at[c].at[pl.ds(s * SLAB, SLAB)]
        )

    return k(table, src_idx, dst_idx, zeros_acc)


def _tc_prep(hist2, x):
    def body(hist2_ref, x_ref, xp_ref, dinv_ref, invc_ref):
        hv = hist2_ref[...]
        hist = (hv[0] + hv[1])[:, 0:1]
        dinv = lax.rsqrt(hist + 1.0)
        invc = 1.0 / jnp.maximum(hist, 1.0)
        dinv_ref[...] = dinv
        invc_ref[...] = invc
        xp_ref[0:N, :] = dinv[0:N] * x_ref[...]
        xp_ref[N:ACC_N, :] = jnp.zeros((ACC_N - N, D), jnp.float32)

    return pl.pallas_call(
        body,
        out_shape=(
            jax.ShapeDtypeStruct((ACC_N, D), jnp.float32),
            jax.ShapeDtypeStruct((ACC_N, 1), jnp.float32),
            jax.ShapeDtypeStruct((ACC_N, 1), jnp.float32),
        ),
    )(hist2, x)


def _tc_layer1(S2, xp, dinv, W1, b1, g1, beta1, Wl):
    def body(S2_ref, xp_ref, dinv_ref, W1_ref, b1_ref, g1_ref, beta1_ref,
             Wl_ref, h2_ref, p_ref):
        Sv = S2_ref[...]
        S = Sv[0, 0:N] + Sv[1, 0:N]
        z = dinv_ref[0:N] * (S + xp_ref[0:N])
        m1 = lax.dot_general(
            z, W1_ref[...], (((1,), (1,)), ((), ())),
            preferred_element_type=jnp.float32,
        ) + b1_ref[...][None, :]
        mu = jnp.mean(m1, axis=0, keepdims=True)
        var = jnp.mean(m1 * m1, axis=0, keepdims=True) - mu * mu
        h2 = (m1 - mu) * lax.rsqrt(var + EPS) * g1_ref[...][None, :] \
            + beta1_ref[...][None, :]
        h2 = jnp.maximum(h2, 0.0)
        h2_ref[...] = h2
        p_ref[0:N, :] = lax.dot_general(
            h2, Wl_ref[...], (((1,), (1,)), ((), ())),
            preferred_element_type=jnp.float32,
        )
        p_ref[N:ACC_N, :] = jnp.zeros((ACC_N - N, D), jnp.float32)

    return pl.pallas_call(
        body,
        out_shape=(
            jax.ShapeDtypeStruct((N, H), jnp.float32),
            jax.ShapeDtypeStruct((ACC_N, D), jnp.float32),
        ),
    )(S2, xp, dinv, W1, b1, g1, beta1, Wl)


def _tc_right(h2, Wr):
    def body(h2_ref, Wr_ref, q_ref):
        q_ref[...] = lax.dot_general(
            h2_ref[...], Wr_ref[...], (((1,), (1,)), ((), ())),
            preferred_element_type=jnp.float32,
        )

    return pl.pallas_call(
        body, out_shape=jax.ShapeDtypeStruct((N, D), jnp.float32)
    )(h2, Wr)


def _tc_layer2(T2, invc, q, bl, g2, beta2):
    def body(T2_ref, invc_ref, q_ref, bl_ref, g2_ref, beta2_ref, out_ref):
        Tv = T2_ref[...]
        T = Tv[0, 0:N] + Tv[1, 0:N]
        r = T * invc_ref[0:N] + bl_ref[...][None, :] + q_ref[...]
        mu = jnp.mean(r, axis=0, keepdims=True)
        var = jnp.mean(r * r, axis=0, keepdims=True) - mu * mu
        out = (r - mu) * lax.rsqrt(var + EPS) * g2_ref[...][None, :] \
            + beta2_ref[...][None, :]
        out_ref[...] = jnp.maximum(out, 0.0)

    return pl.pallas_call(
        body, out_shape=jax.ShapeDtypeStruct((N, D), jnp.float32)
    )(T2, invc, q, bl, g2, beta2)


def kernel(x, edge_index, W1, b1, g1, beta1, Wl, bl, Wr, g2, beta2):
    src = edge_index[0].astype(jnp.int32)
    dst = edge_index[1].astype(jnp.int32)
    pad = E_PAD - E
    src_p = jnp.concatenate([src, jnp.full((pad,), N, jnp.int32)])
    dst_p = jnp.concatenate([dst, jnp.full((pad,), N, jnp.int32)])
    src_p = src_p.reshape(NW, NCHUNK, CHUNK)
    dst_p = dst_p.reshape(NW, NCHUNK, CHUNK)

    ones_rows = jnp.ones((CHUNK, HW), jnp.float32)
    zeros_hist = jnp.zeros((ACC_N, HW), jnp.float32)
    zeros_acc = jnp.zeros((ACC_N, D), jnp.float32)

    hist2 = _sc_hist(dst_p, ones_rows, zeros_hist)
    xp, dinv, invc = _tc_prep(hist2, x)
    S2 = _sc_agg(xp, src_p, dst_p, zeros_acc)
    h2, p = _tc_layer1(S2, xp, dinv, W1, b1, g1, beta1, Wl)
    q = _tc_right(h2, Wr)
    T2 = _sc_agg(p, src_p, dst_p, zeros_acc)
    return _tc_layer2(T2, invc, q, bl, g2, beta2)


# speedup vs baseline: 8.6453x; 1.0967x over previous
import functools

import jax
import jax.numpy as jnp
from jax import lax
from jax.experimental import pallas as pl
from jax.experimental.pallas import tpu as pltpu
from jax.experimental.pallas import tpu_sc as plsc

N = 10000
E = 320000
D = 128
H = 256
NC = 2
NS = 16
NW = NC * NS
CHUNK = 128
NCHUNK = 80
IDXB = 40
PER_TILE = NCHUNK * CHUNK
E_PAD = NW * PER_TILE
ACC_N = 10240
SLAB = ACC_N // NS
HW = 128
EPS = 1e-5

_mesh = functools.partial(
    plsc.VectorSubcoreMesh, core_axis_name="c", subcore_axis_name="s"
)


def _sc_hist(dst_idx, ones_rows, zeros_hist):
    @functools.partial(
        pl.kernel,
        out_type=jax.ShapeDtypeStruct((NC, ACC_N, HW), jnp.float32),
        mesh=_mesh(),
        scratch_types=[
            pltpu.VMEM((NCHUNK, CHUNK), jnp.int32),
            pltpu.VMEM((CHUNK, HW), jnp.float32),
            pltpu.VMEM_SHARED((ACC_N, HW), jnp.float32),
        ],
    )
    def k(dst_hbm, ones_hbm, zeros_hbm, out_hbm, dst_v, ones_v, acc):
        c = lax.axis_index("c")
        s = lax.axis_index("s")
        wid = s * NC + c
        pltpu.sync_copy(
            zeros_hbm.at[pl.ds(s * SLAB, SLAB)], acc.at[pl.ds(s * SLAB, SLAB)]
        )
        pltpu.sync_copy(dst_hbm.at[wid], dst_v)
        pltpu.sync_copy(ones_hbm, ones_v)
        plsc.subcore_barrier()

        @pl.loop(0, NCHUNK)
        def _(j):
            pltpu.sync_copy(ones_v, acc.at[dst_v.at[j]], add=True)

        plsc.subcore_barrier()
        pltpu.sync_copy(
            acc.at[pl.ds(s * SLAB, SLAB)], out_hbm.at[c].at[pl.ds(s * SLAB, SLAB)]
        )

    return k(dst_idx, ones_rows, zeros_hist)


def _sc_agg(table, src_idx, dst_idx, zeros_acc):
    @functools.partial(
        pl.kernel,
        out_type=jax.ShapeDtypeStruct((NC, ACC_N, D), jnp.float32),
        mesh=_mesh(),
        scratch_types=[
            pltpu.VMEM((IDXB, CHUNK), jnp.int32),
            pltpu.VMEM((IDXB, CHUNK), jnp.int32),
            pltpu.VMEM((CHUNK, D), jnp.float32),
            pltpu.VMEM((CHUNK, D), jnp.float32),
            pltpu.VMEM_SHARED((ACC_N, D), jnp.float32),
            pltpu.SemaphoreType.DMA,
            pltpu.SemaphoreType.DMA,
        ],
    )
    def k(table_hbm, src_hbm, dst_hbm, zeros_hbm, out_hbm,
          src_v, dst_v, rows0, rows1, acc, sem0, sem1):
        c = lax.axis_index("c")
        s = lax.axis_index("s")
        wid = s * NC + c
        pltpu.sync_copy(
            zeros_hbm.at[pl.ds(s * SLAB, SLAB)], acc.at[pl.ds(s * SLAB, SLAB)]
        )
        plsc.subcore_barrier()

        def gather(j, buf, sem):
            return pltpu.make_async_copy(table_hbm.at[src_v.at[j]], buf, sem)

        for half in range(NCHUNK // IDXB):
            pltpu.sync_copy(src_hbm.at[wid].at[pl.ds(half * IDXB, IDXB)], src_v)
            pltpu.sync_copy(dst_hbm.at[wid].at[pl.ds(half * IDXB, IDXB)], dst_v)
            gather(0, rows0, sem0).start()
            gather(1, rows1, sem1).start()

            @pl.loop(0, IDXB // 2 - 1)
            def _(i):
                j = 2 * i
                gather(j, rows0, sem0).wait()
                pltpu.sync_copy(rows0, acc.at[dst_v.at[j]], add=True)
                gather(j + 2, rows0, sem0).start()
                gather(j + 1, rows1, sem1).wait()
                pltpu.sync_copy(rows1, acc.at[dst_v.at[j + 1]], add=True)
                gather(j + 3, rows1, sem1).start()

            gather(IDXB - 2, rows0, sem0).wait()
            pltpu.sync_copy(rows0, acc.at[dst_v.at[IDXB - 2]], add=True)
            gather(IDXB - 1, rows1, sem1).wait()
            pltpu.sync_copy(rows1, acc.at[dst_v.at[IDXB - 1]], add=True)

        plsc.subcore_barrier()
        pltpu.sync_copy(
            acc.at[pl.ds(s * SLAB, SLAB)], out_hbm.at[c].at[pl.ds(s * SLAB, SLAB)]
        )

    return k(table, src_idx, dst_idx, zeros_acc)


def _tc_prep(hist2, x):
    def body(hist2_ref, x_ref, xp_ref, dinv_ref, invc_ref):
        hv = hist2_ref[...]
        hist = (hv[0] + hv[1])[:, 0:1]
        dinv = lax.rsqrt(hist + 1.0)
        invc = 1.0 / jnp.maximum(hist, 1.0)
        dinv_ref[...] = dinv
        invc_ref[...] = invc
        xp_ref[0:N, :] = dinv[0:N] * x_ref[...]
        xp_ref[N:ACC_N, :] = jnp.zeros((ACC_N - N, D), jnp.float32)

    return pl.pallas_call(
        body,
        out_shape=(
            jax.ShapeDtypeStruct((ACC_N, D), jnp.float32),
            jax.ShapeDtypeStruct((ACC_N, 1), jnp.float32),
            jax.ShapeDtypeStruct((ACC_N, 1), jnp.float32),
        ),
    )(hist2, x)


def _tc_layer1(S2, xp, dinv, W1, b1, g1, beta1, Wl):
    def body(S2_ref, xp_ref, dinv_ref, W1_ref, b1_ref, g1_ref, beta1_ref,
             Wl_ref, h2_ref, p_ref):
        Sv = S2_ref[...]
        S = Sv[0, 0:N] + Sv[1, 0:N]
        z = dinv_ref[0:N] * (S + xp_ref[0:N])
        m1 = lax.dot_general(
            z, W1_ref[...], (((1,), (1,)), ((), ())),
            preferred_element_type=jnp.float32,
        ) + b1_ref[...][None, :]
        mu = jnp.mean(m1, axis=0, keepdims=True)
        var = jnp.mean(m1 * m1, axis=0, keepdims=True) - mu * mu
        h2 = (m1 - mu) * lax.rsqrt(var + EPS) * g1_ref[...][None, :] \
            + beta1_ref[...][None, :]
        h2 = jnp.maximum(h2, 0.0)
        h2_ref[...] = h2
        p_ref[0:N, :] = lax.dot_general(
            h2, Wl_ref[...], (((1,), (1,)), ((), ())),
            preferred_element_type=jnp.float32,
        )
        p_ref[N:ACC_N, :] = jnp.zeros((ACC_N - N, D), jnp.float32)

    return pl.pallas_call(
        body,
        out_shape=(
            jax.ShapeDtypeStruct((N, H), jnp.float32),
            jax.ShapeDtypeStruct((ACC_N, D), jnp.float32),
        ),
    )(S2, xp, dinv, W1, b1, g1, beta1, Wl)


def _tc_right(h2, Wr):
    def body(h2_ref, Wr_ref, q_ref):
        q_ref[...] = lax.dot_general(
            h2_ref[...], Wr_ref[...], (((1,), (1,)), ((), ())),
            preferred_element_type=jnp.float32,
        )

    return pl.pallas_call(
        body, out_shape=jax.ShapeDtypeStruct((N, D), jnp.float32)
    )(h2, Wr)


def _tc_layer2(T2, invc, q, bl, g2, beta2):
    def body(T2_ref, invc_ref, q_ref, bl_ref, g2_ref, beta2_ref, out_ref):
        Tv = T2_ref[...]
        T = Tv[0, 0:N] + Tv[1, 0:N]
        r = T * invc_ref[0:N] + bl_ref[...][None, :] + q_ref[...]
        mu = jnp.mean(r, axis=0, keepdims=True)
        var = jnp.mean(r * r, axis=0, keepdims=True) - mu * mu
        out = (r - mu) * lax.rsqrt(var + EPS) * g2_ref[...][None, :] \
            + beta2_ref[...][None, :]
        out_ref[...] = jnp.maximum(out, 0.0)

    return pl.pallas_call(
        body, out_shape=jax.ShapeDtypeStruct((N, D), jnp.float32)
    )(T2, invc, q, bl, g2, beta2)


def kernel(x, edge_index, W1, b1, g1, beta1, Wl, bl, Wr, g2, beta2):
    src = edge_index[0].astype(jnp.int32)
    dst = edge_index[1].astype(jnp.int32)
    pad = E_PAD - E
    src_p = jnp.concatenate([src, jnp.full((pad,), N, jnp.int32)])
    dst_p = jnp.concatenate([dst, jnp.full((pad,), N, jnp.int32)])
    src_p = src_p.reshape(NW, NCHUNK, CHUNK)
    dst_p = dst_p.reshape(NW, NCHUNK, CHUNK)

    ones_rows = jnp.ones((CHUNK, HW), jnp.float32)
    zeros_hist = jnp.zeros((ACC_N, HW), jnp.float32)
    zeros_acc = jnp.zeros((ACC_N, D), jnp.float32)

    hist2 = _sc_hist(dst_p, ones_rows, zeros_hist)
    xp, dinv, invc = _tc_prep(hist2, x)
    S2 = _sc_agg(xp, src_p, dst_p, zeros_acc)
    h2, p = _tc_layer1(S2, xp, dinv, W1, b1, g1, beta1, Wl)
    q = _tc_right(h2, Wr)
    T2 = _sc_agg(p, src_p, dst_p, zeros_acc)
    return _tc_layer2(T2, invc, q, bl, g2, beta2)


# speedup vs baseline: 26.1687x; 3.0269x over previous
import functools

import jax
import jax.numpy as jnp
from jax import lax
from jax.experimental import pallas as pl
from jax.experimental.pallas import tpu as pltpu
from jax.experimental.pallas import tpu_sc as plsc

N = 10000
E = 320000
D = 128
H = 256
NC = 2
NS = 16
NW = NC * NS
CHUNK = 128
NCHUNK = 80
IDXB = 40
PER_TILE = NCHUNK * CHUNK
E_PAD = NW * PER_TILE
ACC_N = 10240
SLAB = ACC_N // NS
HW = 128
EPS = 1e-5

_mesh = functools.partial(
    plsc.VectorSubcoreMesh, core_axis_name="c", subcore_axis_name="s"
)


def _sc_hist(dst_idx, ones_rows, zeros_hist):
    @functools.partial(
        pl.kernel,
        out_type=jax.ShapeDtypeStruct((NC, ACC_N, HW), jnp.float32),
        mesh=_mesh(),
        scratch_types=[
            pltpu.VMEM((NCHUNK, CHUNK), jnp.int32),
            pltpu.VMEM((CHUNK, HW), jnp.float32),
            pltpu.VMEM_SHARED((ACC_N, HW), jnp.float32),
        ],
    )
    def k(dst_hbm, ones_hbm, zeros_hbm, out_hbm, dst_v, ones_v, acc):
        c = lax.axis_index("c")
        s = lax.axis_index("s")
        wid = s * NC + c
        pltpu.sync_copy(
            zeros_hbm.at[pl.ds(s * SLAB, SLAB)], acc.at[pl.ds(s * SLAB, SLAB)]
        )
        pltpu.sync_copy(dst_hbm.at[wid], dst_v)
        pltpu.sync_copy(ones_hbm, ones_v)
        plsc.subcore_barrier()

        @pl.loop(0, NCHUNK)
        def _(j):
            pltpu.sync_copy(ones_v, acc.at[dst_v.at[j]], add=True)

        plsc.subcore_barrier()
        pltpu.sync_copy(
            acc.at[pl.ds(s * SLAB, SLAB)], out_hbm.at[c].at[pl.ds(s * SLAB, SLAB)]
        )

    return k(dst_idx, ones_rows, zeros_hist)


def _sc_agg(table, src_idx, dst_idx, zeros_acc):
    @functools.partial(
        pl.kernel,
        out_type=jax.ShapeDtypeStruct((NC, ACC_N, D), jnp.float32),
        mesh=_mesh(),
        scratch_types=[
            pltpu.VMEM((IDXB, CHUNK), jnp.int32),
            pltpu.VMEM((IDXB, CHUNK), jnp.int32),
            pltpu.VMEM((CHUNK, D), jnp.float32),
            pltpu.VMEM((CHUNK, D), jnp.float32),
            pltpu.VMEM_SHARED((ACC_N, D), jnp.float32),
            pltpu.SemaphoreType.DMA,
            pltpu.SemaphoreType.DMA,
        ],
    )
    def k(table_hbm, src_hbm, dst_hbm, zeros_hbm, out_hbm,
          src_v, dst_v, rows0, rows1, acc, sem0, sem1):
        c = lax.axis_index("c")
        s = lax.axis_index("s")
        wid = s * NC + c
        pltpu.sync_copy(
            zeros_hbm.at[pl.ds(s * SLAB, SLAB)], acc.at[pl.ds(s * SLAB, SLAB)]
        )
        plsc.subcore_barrier()

        def gather(j, buf, sem):
            return pltpu.make_async_copy(table_hbm.at[src_v.at[j]], buf, sem)

        for half in range(NCHUNK // IDXB):
            pltpu.sync_copy(src_hbm.at[wid].at[pl.ds(half * IDXB, IDXB)], src_v)
            pltpu.sync_copy(dst_hbm.at[wid].at[pl.ds(half * IDXB, IDXB)], dst_v)
            gather(0, rows0, sem0).start()
            gather(1, rows1, sem1).start()

            @pl.loop(0, IDXB // 2 - 1)
            def _(i):
                j = 2 * i
                gather(j, rows0, sem0).wait()
                pltpu.sync_copy(rows0, acc.at[dst_v.at[j]], add=True)
                gather(j + 2, rows0, sem0).start()
                gather(j + 1, rows1, sem1).wait()
                pltpu.sync_copy(rows1, acc.at[dst_v.at[j + 1]], add=True)
                gather(j + 3, rows1, sem1).start()

            gather(IDXB - 2, rows0, sem0).wait()
            pltpu.sync_copy(rows0, acc.at[dst_v.at[IDXB - 2]], add=True)
            gather(IDXB - 1, rows1, sem1).wait()
            pltpu.sync_copy(rows1, acc.at[dst_v.at[IDXB - 1]], add=True)

        plsc.subcore_barrier()
        pltpu.sync_copy(
            acc.at[pl.ds(s * SLAB, SLAB)], out_hbm.at[c].at[pl.ds(s * SLAB, SLAB)]
        )

    return k(table, src_idx, dst_idx, zeros_acc)


def _tc_prep(hist2, x):
    def body(hist2_ref, x_ref, xp_ref, dinv_ref, invc_ref):
        hv = hist2_ref[...]
        hist = (hv[0] + hv[1])[:, 0:1]
        dinv = lax.rsqrt(hist + 1.0)
        invc = 1.0 / jnp.maximum(hist, 1.0)
        dinv_ref[...] = dinv
        invc_ref[...] = invc
        xp_ref[0:N, :] = dinv[0:N] * x_ref[...]
        xp_ref[N:ACC_N, :] = jnp.zeros((ACC_N - N, D), jnp.float32)

    return pl.pallas_call(
        body,
        out_shape=(
            jax.ShapeDtypeStruct((ACC_N, D), jnp.float32),
            jax.ShapeDtypeStruct((ACC_N, 1), jnp.float32),
            jax.ShapeDtypeStruct((ACC_N, 1), jnp.float32),
        ),
    )(hist2, x)


def _tc_layer1(S2, xp, dinv, W1, b1, g1, beta1, Wl):
    def body(S2_ref, xp_ref, dinv_ref, W1_ref, b1_ref, g1_ref, beta1_ref,
             Wl_ref, h2_ref, p_ref):
        Sv = S2_ref[...]
        S = Sv[0, 0:N] + Sv[1, 0:N]
        z = dinv_ref[0:N] * (S + xp_ref[0:N])
        m1 = lax.dot_general(
            z, W1_ref[...], (((1,), (1,)), ((), ())),
            preferred_element_type=jnp.float32,
        ) + b1_ref[...][None, :]
        mu = jnp.mean(m1, axis=0, keepdims=True)
        var = jnp.mean(m1 * m1, axis=0, keepdims=True) - mu * mu
        h2 = (m1 - mu) * lax.rsqrt(var + EPS) * g1_ref[...][None, :] \
            + beta1_ref[...][None, :]
        h2 = jnp.maximum(h2, 0.0)
        h2_ref[...] = h2
        p_ref[0:N, :] = lax.dot_general(
            h2, Wl_ref[...], (((1,), (1,)), ((), ())),
            preferred_element_type=jnp.float32,
        )
        p_ref[N:ACC_N, :] = jnp.zeros((ACC_N - N, D), jnp.float32)

    return pl.pallas_call(
        body,
        out_shape=(
            jax.ShapeDtypeStruct((N, H), jnp.float32),
            jax.ShapeDtypeStruct((ACC_N, D), jnp.float32),
        ),
    )(S2, xp, dinv, W1, b1, g1, beta1, Wl)


def _tc_right(h2, Wr):
    def body(h2_ref, Wr_ref, q_ref):
        q_ref[...] = lax.dot_general(
            h2_ref[...], Wr_ref[...], (((1,), (1,)), ((), ())),
            preferred_element_type=jnp.float32,
        )

    return pl.pallas_call(
        body, out_shape=jax.ShapeDtypeStruct((N, D), jnp.float32)
    )(h2, Wr)


def _tc_layer2(T2, invc, q, bl, g2, beta2):
    def body(T2_ref, invc_ref, q_ref, bl_ref, g2_ref, beta2_ref, out_ref):
        Tv = T2_ref[...]
        T = Tv[0, 0:N] + Tv[1, 0:N]
        r = T * invc_ref[0:N] + bl_ref[...][None, :] + q_ref[...]
        mu = jnp.mean(r, axis=0, keepdims=True)
        var = jnp.mean(r * r, axis=0, keepdims=True) - mu * mu
        out = (r - mu) * lax.rsqrt(var + EPS) * g2_ref[...][None, :] \
            + beta2_ref[...][None, :]
        out_ref[...] = jnp.maximum(out, 0.0)

    return pl.pallas_call(
        body, out_shape=jax.ShapeDtypeStruct((N, D), jnp.float32)
    )(T2, invc, q, bl, g2, beta2)


def kernel(x, edge_index, W1, b1, g1, beta1, Wl, bl, Wr, g2, beta2):
    src = edge_index[0].astype(jnp.int32)
    dst = edge_index[1].astype(jnp.int32)
    pad = E_PAD - E
    dummy = N + (jnp.arange(pad, dtype=jnp.int32) % (ACC_N - N))
    src_p = jnp.concatenate([src, dummy])
    dst_p = jnp.concatenate([dst, dummy])
    src_p = src_p.reshape(NCHUNK * CHUNK, NW).T.reshape(NW, NCHUNK, CHUNK)
    dst_p = dst_p.reshape(NCHUNK * CHUNK, NW).T.reshape(NW, NCHUNK, CHUNK)

    ones_rows = jnp.ones((CHUNK, HW), jnp.float32)
    zeros_hist = jnp.zeros((ACC_N, HW), jnp.float32)
    zeros_acc = jnp.zeros((ACC_N, D), jnp.float32)

    hist2 = _sc_hist(dst_p, ones_rows, zeros_hist)
    xp, dinv, invc = _tc_prep(hist2, x)
    S2 = _sc_agg(xp, src_p, dst_p, zeros_acc)
    h2, p = _tc_layer1(S2, xp, dinv, W1, b1, g1, beta1, Wl)
    q = _tc_right(h2, Wr)
    T2 = _sc_agg(p, src_p, dst_p, zeros_acc)
    return _tc_layer2(T2, invc, q, bl, g2, beta2)


# speedup vs baseline: 31.5306x; 1.2049x over previous
import dataclasses
import functools

import jax
import jax.numpy as jnp
from jax import lax
from jax.experimental import pallas as pl
from jax.experimental.pallas import tpu as pltpu
from jax.experimental.pallas import tpu_sc as plsc

N = 10000
E = 320000
D = 128
H = 256
NC = 2
NS = 16
NW = NC * NS
CHUNK = 128
NCHUNK = 80
IDXB = 40
PER_TILE = NCHUNK * CHUNK
E_PAD = NW * PER_TILE
ACC_N = 10240
SLAB = ACC_N // NS
HW = 128
EPS = 1e-5

_mesh = functools.partial(
    plsc.VectorSubcoreMesh, core_axis_name="c", subcore_axis_name="s"
)


def _sc_hist(dst_flat):
    cp = pltpu.CompilerParams()
    if "needs_layout_passes" in pltpu.CompilerParams.__dataclass_fields__:
        cp = dataclasses.replace(cp, needs_layout_passes=False)

    @functools.partial(
        pl.kernel,
        out_type=jax.ShapeDtypeStruct((NW, ACC_N), jnp.float32),
        mesh=_mesh(),
        compiler_params=cp,
        scratch_types=[
            pltpu.VMEM((PER_TILE,), jnp.int32),
            pltpu.VMEM((ACC_N,), jnp.float32),
        ],
    )
    def k(dst_hbm, out_hbm, dst_v, hist_v):
        c = lax.axis_index("c")
        s = lax.axis_index("s")
        wid = s * NC + c
        pltpu.sync_copy(dst_hbm.at[wid], dst_v)
        zeros16 = jnp.zeros((16,), jnp.float32)

        @pl.loop(0, ACC_N // 16)
        def _(i):
            hist_v[pl.ds(i * 16, 16)] = zeros16

        ones16 = jnp.ones((16,), jnp.float32)

        @pl.loop(0, PER_TILE // 16)
        def _(i):
            iv = dst_v[pl.ds(i * 16, 16)]
            plsc.addupdate_scatter(hist_v, [iv], ones16)

        pltpu.sync_copy(hist_v, out_hbm.at[wid])

    return k(dst_flat)


def _sc_agg(table, src_idx, dst_idx, zeros_acc):
    @functools.partial(
        pl.kernel,
        out_type=jax.ShapeDtypeStruct((NC, ACC_N, D), jnp.float32),
        mesh=_mesh(),
        scratch_types=[
            pltpu.VMEM((IDXB, CHUNK), jnp.int32),
            pltpu.VMEM((IDXB, CHUNK), jnp.int32),
            pltpu.VMEM((CHUNK, D), jnp.float32),
            pltpu.VMEM((CHUNK, D), jnp.float32),
            pltpu.VMEM_SHARED((ACC_N, D), jnp.float32),
            pltpu.SemaphoreType.DMA,
            pltpu.SemaphoreType.DMA,
        ],
    )
    def k(table_hbm, src_hbm, dst_hbm, zeros_hbm, out_hbm,
          src_v, dst_v, rows0, rows1, acc, sem0, sem1):
        c = lax.axis_index("c")
        s = lax.axis_index("s")
        wid = s * NC + c
        pltpu.sync_copy(
            zeros_hbm.at[pl.ds(s * SLAB, SLAB)], acc.at[pl.ds(s * SLAB, SLAB)]
        )
        plsc.subcore_barrier()

        def gather(j, buf, sem):
            return pltpu.make_async_copy(table_hbm.at[src_v.at[j]], buf, sem)

        for half in range(NCHUNK // IDXB):
            pltpu.sync_copy(src_hbm.at[wid].at[pl.ds(half * IDXB, IDXB)], src_v)
            pltpu.sync_copy(dst_hbm.at[wid].at[pl.ds(half * IDXB, IDXB)], dst_v)
            gather(0, rows0, sem0).start()
            gather(1, rows1, sem1).start()

            @pl.loop(0, IDXB // 2 - 1)
            def _(i):
                j = 2 * i
                gather(j, rows0, sem0).wait()
                pltpu.sync_copy(rows0, acc.at[dst_v.at[j]], add=True)
                gather(j + 2, rows0, sem0).start()
                gather(j + 1, rows1, sem1).wait()
                pltpu.sync_copy(rows1, acc.at[dst_v.at[j + 1]], add=True)
                gather(j + 3, rows1, sem1).start()

            gather(IDXB - 2, rows0, sem0).wait()
            pltpu.sync_copy(rows0, acc.at[dst_v.at[IDXB - 2]], add=True)
            gather(IDXB - 1, rows1, sem1).wait()
            pltpu.sync_copy(rows1, acc.at[dst_v.at[IDXB - 1]], add=True)

        plsc.subcore_barrier()
        pltpu.sync_copy(
            acc.at[pl.ds(s * SLAB, SLAB)], out_hbm.at[c].at[pl.ds(s * SLAB, SLAB)]
        )

    return k(table, src_idx, dst_idx, zeros_acc)


def _tc_prep(hist2, x):
    def body(hist2_ref, x_ref, xp_ref, dinv_ref, invc_ref):
        hv = hist2_ref[...]
        hist = jnp.sum(hv, axis=0)[:, None]
        dinv = lax.rsqrt(hist + 1.0)
        invc = 1.0 / jnp.maximum(hist, 1.0)
        dinv_ref[...] = dinv
        invc_ref[...] = invc
        xp_ref[0:N, :] = dinv[0:N] * x_ref[...]
        xp_ref[N:ACC_N, :] = jnp.zeros((ACC_N - N, D), jnp.float32)

    return pl.pallas_call(
        body,
        out_shape=(
            jax.ShapeDtypeStruct((ACC_N, D), jnp.float32),
            jax.ShapeDtypeStruct((ACC_N, 1), jnp.float32),
            jax.ShapeDtypeStruct((ACC_N, 1), jnp.float32),
        ),
    )(hist2, x)


def _tc_layer1(S2, xp, dinv, W1, b1, g1, beta1, Wl):
    def body(S2_ref, xp_ref, dinv_ref, W1_ref, b1_ref, g1_ref, beta1_ref,
             Wl_ref, h2_ref, p_ref):
        Sv = S2_ref[...]
        S = Sv[0, 0:N] + Sv[1, 0:N]
        z = dinv_ref[0:N] * (S + xp_ref[0:N])
        m1 = lax.dot_general(
            z, W1_ref[...], (((1,), (1,)), ((), ())),
            preferred_element_type=jnp.float32,
        ) + b1_ref[...][None, :]
        mu = jnp.mean(m1, axis=0, keepdims=True)
        var = jnp.mean(m1 * m1, axis=0, keepdims=True) - mu * mu
        h2 = (m1 - mu) * lax.rsqrt(var + EPS) * g1_ref[...][None, :] \
            + beta1_ref[...][None, :]
        h2 = jnp.maximum(h2, 0.0)
        h2_ref[...] = h2
        p_ref[0:N, :] = lax.dot_general(
            h2, Wl_ref[...], (((1,), (1,)), ((), ())),
            preferred_element_type=jnp.float32,
        )
        p_ref[N:ACC_N, :] = jnp.zeros((ACC_N - N, D), jnp.float32)

    return pl.pallas_call(
        body,
        out_shape=(
            jax.ShapeDtypeStruct((N, H), jnp.float32),
            jax.ShapeDtypeStruct((ACC_N, D), jnp.float32),
        ),
    )(S2, xp, dinv, W1, b1, g1, beta1, Wl)


def _tc_right(h2, Wr):
    def body(h2_ref, Wr_ref, q_ref):
        q_ref[...] = lax.dot_general(
            h2_ref[...], Wr_ref[...], (((1,), (1,)), ((), ())),
            preferred_element_type=jnp.float32,
        )

    return pl.pallas_call(
        body, out_shape=jax.ShapeDtypeStruct((N, D), jnp.float32)
    )(h2, Wr)


def _tc_layer2(T2, invc, q, bl, g2, beta2):
    def body(T2_ref, invc_ref, q_ref, bl_ref, g2_ref, beta2_ref, out_ref):
        Tv = T2_ref[...]
        T = Tv[0, 0:N] + Tv[1, 0:N]
        r = T * invc_ref[0:N] + bl_ref[...][None, :] + q_ref[...]
        mu = jnp.mean(r, axis=0, keepdims=True)
        var = jnp.mean(r * r, axis=0, keepdims=True) - mu * mu
        out = (r - mu) * lax.rsqrt(var + EPS) * g2_ref[...][None, :] \
            + beta2_ref[...][None, :]
        out_ref[...] = jnp.maximum(out, 0.0)

    return pl.pallas_call(
        body, out_shape=jax.ShapeDtypeStruct((N, D), jnp.float32)
    )(T2, invc, q, bl, g2, beta2)


def kernel(x, edge_index, W1, b1, g1, beta1, Wl, bl, Wr, g2, beta2):
    src = edge_index[0].astype(jnp.int32)
    dst = edge_index[1].astype(jnp.int32)
    pad = E_PAD - E
    dummy = N + (jnp.arange(pad, dtype=jnp.int32) % (ACC_N - N))
    src_p = jnp.concatenate([src, dummy])
    dst_p = jnp.concatenate([dst, dummy])
    src_p = src_p.reshape(NCHUNK * CHUNK, NW).T.reshape(NW, NCHUNK, CHUNK)
    dst_p = dst_p.reshape(NCHUNK * CHUNK, NW).T.reshape(NW, NCHUNK, CHUNK)

    zeros_acc = jnp.zeros((ACC_N, D), jnp.float32)

    hist2 = _sc_hist(dst_p.reshape(NW, PER_TILE))
    xp, dinv, invc = _tc_prep(hist2, x)
    S2 = _sc_agg(xp, src_p, dst_p, zeros_acc)
    h2, p = _tc_layer1(S2, xp, dinv, W1, b1, g1, beta1, Wl)
    q = _tc_right(h2, Wr)
    T2 = _sc_agg(p, src_p, dst_p, zeros_acc)
    return _tc_layer2(T2, invc, q, bl, g2, beta2)


# speedup vs baseline: 32.5547x; 1.0325x over previous
import dataclasses
import functools

import jax
import jax.numpy as jnp
from jax import lax
from jax.experimental import pallas as pl
from jax.experimental.pallas import tpu as pltpu
from jax.experimental.pallas import tpu_sc as plsc

N = 10000
E = 320000
D = 128
H = 256
NC = 2
NS = 16
NW = NC * NS
CHUNK = 128
NCHUNK = 80
IDXB = 40
PER_TILE = NCHUNK * CHUNK
E_PAD = NW * PER_TILE
ACC_N = 10240
SLAB = ACC_N // NS
HW = 128
EPS = 1e-5

_mesh = functools.partial(
    plsc.VectorSubcoreMesh, core_axis_name="c", subcore_axis_name="s"
)


def _sc_hist(dst_flat):
    cp = pltpu.CompilerParams()
    if "needs_layout_passes" in pltpu.CompilerParams.__dataclass_fields__:
        cp = dataclasses.replace(cp, needs_layout_passes=False)

    @functools.partial(
        pl.kernel,
        out_type=jax.ShapeDtypeStruct((NW, ACC_N), jnp.float32),
        mesh=_mesh(),
        compiler_params=cp,
        scratch_types=[
            pltpu.VMEM((PER_TILE,), jnp.int32),
            pltpu.VMEM((ACC_N,), jnp.float32),
        ],
    )
    def k(dst_hbm, out_hbm, dst_v, hist_v):
        c = lax.axis_index("c")
        s = lax.axis_index("s")
        wid = s * NC + c
        pltpu.sync_copy(dst_hbm.at[wid], dst_v)
        zeros16 = jnp.zeros((16,), jnp.float32)

        @pl.loop(0, ACC_N // 16)
        def _(i):
            hist_v[pl.ds(i * 16, 16)] = zeros16

        ones16 = jnp.ones((16,), jnp.float32)

        @pl.loop(0, PER_TILE // 16)
        def _(i):
            iv = dst_v[pl.ds(i * 16, 16)]
            plsc.addupdate_scatter(hist_v, [iv], ones16)

        pltpu.sync_copy(hist_v, out_hbm.at[wid])

    return k(dst_flat)


def _sc_agg(table, src_idx, dst_idx, zeros_acc):
    @functools.partial(
        pl.kernel,
        out_type=jax.ShapeDtypeStruct((NC, ACC_N, D), jnp.float32),
        mesh=_mesh(),
        scratch_types=[
            pltpu.VMEM((IDXB, CHUNK), jnp.int32),
            pltpu.VMEM((IDXB, CHUNK), jnp.int32),
            pltpu.VMEM((CHUNK, D), jnp.float32),
            pltpu.VMEM((CHUNK, D), jnp.float32),
            pltpu.VMEM_SHARED((ACC_N, D), jnp.float32),
            pltpu.SemaphoreType.DMA,
            pltpu.SemaphoreType.DMA,
        ],
    )
    def k(table_hbm, src_hbm, dst_hbm, zeros_hbm, out_hbm,
          src_v, dst_v, rows0, rows1, acc, sem0, sem1):
        c = lax.axis_index("c")
        s = lax.axis_index("s")
        wid = s * NC + c
        pltpu.sync_copy(
            zeros_hbm.at[pl.ds(s * SLAB, SLAB)], acc.at[pl.ds(s * SLAB, SLAB)]
        )
        plsc.subcore_barrier()

        def gather(j, buf, sem):
            return pltpu.make_async_copy(table_hbm.at[src_v.at[j]], buf, sem)

        for half in range(NCHUNK // IDXB):
            pltpu.sync_copy(src_hbm.at[wid].at[pl.ds(half * IDXB, IDXB)], src_v)
            pltpu.sync_copy(dst_hbm.at[wid].at[pl.ds(half * IDXB, IDXB)], dst_v)
            gather(0, rows0, sem0).start()
            gather(1, rows1, sem1).start()

            @pl.loop(0, IDXB // 2 - 1)
            def _(i):
                j = 2 * i
                gather(j, rows0, sem0).wait()
                pltpu.sync_copy(rows0, acc.at[dst_v.at[j]], add=True)
                gather(j + 2, rows0, sem0).start()
                gather(j + 1, rows1, sem1).wait()
                pltpu.sync_copy(rows1, acc.at[dst_v.at[j + 1]], add=True)
                gather(j + 3, rows1, sem1).start()

            gather(IDXB - 2, rows0, sem0).wait()
            pltpu.sync_copy(rows0, acc.at[dst_v.at[IDXB - 2]], add=True)
            gather(IDXB - 1, rows1, sem1).wait()
            pltpu.sync_copy(rows1, acc.at[dst_v.at[IDXB - 1]], add=True)

        plsc.subcore_barrier()
        pltpu.sync_copy(
            acc.at[pl.ds(s * SLAB, SLAB)], out_hbm.at[c].at[pl.ds(s * SLAB, SLAB)]
        )

    return k(table, src_idx, dst_idx, zeros_acc)


def _tc_prep(hist2, x):
    def body(hist2_ref, x_ref, xp_ref, dinv_ref, invc_ref):
        hv = hist2_ref[...]
        hist = jnp.sum(hv, axis=0)[:, None]
        dinv = lax.rsqrt(hist + 1.0)
        invc = 1.0 / jnp.maximum(hist, 1.0)
        dinv_ref[...] = dinv
        invc_ref[...] = invc
        xp_ref[0:N, :] = dinv[0:N] * x_ref[...]
        xp_ref[N:ACC_N, :] = jnp.zeros((ACC_N - N, D), jnp.float32)

    return pl.pallas_call(
        body,
        out_shape=(
            jax.ShapeDtypeStruct((ACC_N, D), jnp.float32),
            jax.ShapeDtypeStruct((ACC_N, 1), jnp.float32),
            jax.ShapeDtypeStruct((ACC_N, 1), jnp.float32),
        ),
    )(hist2, x)


def _tc_layer1(S2, xp, dinv, W1, b1, g1, beta1, Wl):
    def body(S2_ref, xp_ref, dinv_ref, W1_ref, b1_ref, g1_ref, beta1_ref,
             Wl_ref, h2_ref, p_ref):
        Sv = S2_ref[...]
        S = Sv[0, 0:N] + Sv[1, 0:N]
        z = dinv_ref[0:N] * (S + xp_ref[0:N])
        m1 = lax.dot_general(
            z, W1_ref[...], (((1,), (1,)), ((), ())),
            preferred_element_type=jnp.float32,
        ) + b1_ref[...][None, :]
        mu = jnp.mean(m1, axis=0, keepdims=True)
        var = jnp.mean(m1 * m1, axis=0, keepdims=True) - mu * mu
        h2 = (m1 - mu) * lax.rsqrt(var + EPS) * g1_ref[...][None, :] \
            + beta1_ref[...][None, :]
        h2 = jnp.maximum(h2, 0.0)
        h2_ref[...] = h2
        p_ref[0:N, :] = lax.dot_general(
            h2, Wl_ref[...], (((1,), (1,)), ((), ())),
            preferred_element_type=jnp.float32,
        )
        p_ref[N:ACC_N, :] = jnp.zeros((ACC_N - N, D), jnp.float32)

    return pl.pallas_call(
        body,
        out_shape=(
            jax.ShapeDtypeStruct((N, H), jnp.float32),
            jax.ShapeDtypeStruct((ACC_N, D), jnp.float32),
        ),
    )(S2, xp, dinv, W1, b1, g1, beta1, Wl)


def _tc_right(h2, Wr):
    def body(h2_ref, Wr_ref, q_ref):
        q_ref[...] = lax.dot_general(
            h2_ref[...], Wr_ref[...], (((1,), (1,)), ((), ())),
            preferred_element_type=jnp.float32,
        )

    return pl.pallas_call(
        body, out_shape=jax.ShapeDtypeStruct((N, D), jnp.float32)
    )(h2, Wr)


def _tc_layer2(T2, invc, q, bl, g2, beta2):
    def body(T2_ref, invc_ref, q_ref, bl_ref, g2_ref, beta2_ref, out_ref):
        Tv = T2_ref[...]
        T = Tv[0, 0:N] + Tv[1, 0:N]
        r = T * invc_ref[0:N] + bl_ref[...][None, :] + q_ref[...]
        mu = jnp.mean(r, axis=0, keepdims=True)
        var = jnp.mean(r * r, axis=0, keepdims=True) - mu * mu
        out = (r - mu) * lax.rsqrt(var + EPS) * g2_ref[...][None, :] \
            + beta2_ref[...][None, :]
        out_ref[...] = jnp.maximum(out, 0.0)

    return pl.pallas_call(
        body, out_shape=jax.ShapeDtypeStruct((N, D), jnp.float32)
    )(T2, invc, q, bl, g2, beta2)


def kernel(x, edge_index, W1, b1, g1, beta1, Wl, bl, Wr, g2, beta2):
    src = edge_index[0].astype(jnp.int32)
    dst = edge_index[1].astype(jnp.int32)
    pad_tile = PER_TILE - E // NW
    dummy = jnp.broadcast_to(
        N + jnp.arange(pad_tile, dtype=jnp.int32), (NW, pad_tile)
    )
    src_p = jnp.concatenate([src.reshape(NW, E // NW), dummy], axis=1)
    dst_p = jnp.concatenate([dst.reshape(NW, E // NW), dummy], axis=1)
    src_p = src_p.reshape(NW, NCHUNK, CHUNK)
    dst_p = dst_p.reshape(NW, NCHUNK, CHUNK)

    zeros_acc = jnp.zeros((ACC_N, D), jnp.float32)

    hist2 = _sc_hist(dst_p.reshape(NW, PER_TILE))
    xp, dinv, invc = _tc_prep(hist2, x)
    S2 = _sc_agg(xp, src_p, dst_p, zeros_acc)
    h2, p = _tc_layer1(S2, xp, dinv, W1, b1, g1, beta1, Wl)
    q = _tc_right(h2, Wr)
    T2 = _sc_agg(p, src_p, dst_p, zeros_acc)
    return _tc_layer2(T2, invc, q, bl, g2, beta2)
